# baseline (speedup 1.0000x reference)
import jax
import jax.numpy as jnp
from jax.experimental import pallas as pl
from jax.experimental.pallas import tpu as pltpu

B = 16
M = 2048
D = 256
H = 8
A = 64
O = 64
HA = H * A
HO = H * O
BPG = 4
NEG = -1e30


def _body(xf_ref, mask_ref, Ws_ref, bs_ref, Wq_ref, bq_ref, Wk_ref, bk_ref,
          Wv_ref, bv_ref, out_ref, frac_ref):
    xf = xf_ref[...].reshape(BPG * M, D)
    activ_b = jnp.dot(xf, Ws_ref[...],
                      preferred_element_type=jnp.float32).astype(jnp.bfloat16)
    activ_b = activ_b + bs_ref[...]
    q = jnp.dot(activ_b, Wq_ref[...],
                preferred_element_type=jnp.float32).astype(jnp.bfloat16)
    k = jnp.dot(activ_b, Wk_ref[...],
                preferred_element_type=jnp.float32).astype(jnp.bfloat16)

    qk = (q + bq_ref[...]) * (k + bk_ref[...])
    lane = jax.lax.broadcasted_iota(jnp.int32, (HA, H), 0)
    head = jax.lax.broadcasted_iota(jnp.int32, (HA, H), 1)
    sel = (lane // A == head).astype(jnp.bfloat16)
    scores = jnp.dot(qk, sel,
                     preferred_element_type=jnp.float32) * (1.0 / (A ** 0.5))

    m = mask_ref[...].reshape(BPG * M, 1)
    s = jnp.where(m > 0.5, scores, NEG)

    row = jax.lax.broadcasted_iota(jnp.int32, (H, HO), 0)
    col = jax.lax.broadcasted_iota(jnp.int32, (H, HO), 1)
    pick = (col // O == row).astype(jnp.float32)

    for i in range(BPG):
        si = s[i * M:(i + 1) * M]
        zmax = jnp.maximum(jnp.max(si, axis=0, keepdims=True), 0.0)
        ez = jnp.exp(si - zmax)
        den = jnp.sum(ez, axis=0, keepdims=True) + 1.0
        attn = (ez / den).astype(jnp.bfloat16)
        ai = activ_b[i * M:(i + 1) * M]
        ta = jax.lax.dot_general(attn, ai, (((0,), (0,)), ((), ())),
                                 preferred_element_type=jnp.float32)
        full = jnp.dot(ta.astype(jnp.bfloat16), Wv_ref[...],
                       preferred_element_type=jnp.float32)
        sa = ((den - 1.0) / den).reshape(H, 1)
        full = full + sa * bv_ref[...]
        out_ref[i] = jnp.sum(full * pick, axis=0, keepdims=True)
        frac_ref[i] = jnp.sum(m[i * M:(i + 1) * M], axis=0,
                              keepdims=True) * (1.0 / M)


@jax.jit
def kernel(x, Ws, bs, Wq, bq, Wk, bk, Wv, bv):
    xf = x[:, : M * D].reshape(B, M, D).astype(jnp.bfloat16)
    mask = x[:, M * D:].reshape(B, M, 1)
    out_main, frac = pl.pallas_call(
        _body,
        grid=(B // BPG,),
        in_specs=[
            pl.BlockSpec((BPG, M, D), lambda b: (b, 0, 0)),
            pl.BlockSpec((BPG, M, 1), lambda b: (b, 0, 0)),
            pl.BlockSpec((D, D), lambda b: (0, 0)),
            pl.BlockSpec((1, D), lambda b: (0, 0)),
            pl.BlockSpec((D, HA), lambda b: (0, 0)),
            pl.BlockSpec((1, HA), lambda b: (0, 0)),
            pl.BlockSpec((D, HA), lambda b: (0, 0)),
            pl.BlockSpec((1, HA), lambda b: (0, 0)),
            pl.BlockSpec((D, HO), lambda b: (0, 0)),
            pl.BlockSpec((1, HO), lambda b: (0, 0)),
        ],
        out_specs=[
            pl.BlockSpec((BPG, 1, HO), lambda b: (b, 0, 0)),
            pl.BlockSpec((BPG, 1, 1), lambda b: (b, 0, 0)),
        ],
        out_shape=[
            jax.ShapeDtypeStruct((B, 1, HO), jnp.float32),
            jax.ShapeDtypeStruct((B, 1, 1), jnp.float32),
        ],
    )(xf, mask, Ws.astype(jnp.bfloat16), bs.astype(jnp.bfloat16).reshape(1, D),
      Wq.astype(jnp.bfloat16), bq.astype(jnp.bfloat16).reshape(1, HA),
      Wk.astype(jnp.bfloat16), bk.astype(jnp.bfloat16).reshape(1, HA),
      Wv.astype(jnp.bfloat16), bv.reshape(1, HO))
    return jnp.concatenate([out_main.reshape(B, HO), frac.reshape(B, 1)],
                           axis=1)


# speedup vs baseline: 1.0181x; 1.0181x over previous
import jax
import jax.numpy as jnp
from jax.experimental import pallas as pl
from jax.experimental.pallas import tpu as pltpu

B = 16
M = 2048
D = 256
H = 8
A = 64
O = 64
HA = H * A
HO = H * O
BPG = 4
NEG = -1e30


def _body(xf_ref, mask_ref, Ws_ref, bs_ref, Wq_ref, bq_ref, Wk_ref, bk_ref,
          Wv_ref, bv_ref, out_ref, frac_ref):
    xf = xf_ref[...].reshape(BPG * M, D)
    activ = jnp.dot(xf, Ws_ref[...],
                    preferred_element_type=jnp.float32) + bs_ref[...]
    activ_b = activ.astype(jnp.bfloat16)
    q = jnp.dot(activ_b, Wq_ref[...],
                preferred_element_type=jnp.float32) + bq_ref[...]
    k = jnp.dot(activ_b, Wk_ref[...],
                preferred_element_type=jnp.float32) + bk_ref[...]

    qk = (q * k).astype(jnp.bfloat16)
    lane = jax.lax.broadcasted_iota(jnp.int32, (HA, H), 0)
    head = jax.lax.broadcasted_iota(jnp.int32, (HA, H), 1)
    sel = (lane // A == head).astype(jnp.bfloat16)
    scores = jnp.dot(qk, sel,
                     preferred_element_type=jnp.float32) * (1.0 / (A ** 0.5))

    m = mask_ref[...].reshape(BPG * M, 1)
    s = jnp.where(m > 0.5, scores, NEG)

    row = jax.lax.broadcasted_iota(jnp.int32, (H, HO), 0)
    col = jax.lax.broadcasted_iota(jnp.int32, (H, HO), 1)
    pick = (col // O == row).astype(jnp.float32)

    for i in range(BPG):
        si = s[i * M:(i + 1) * M]
        zmax = jnp.maximum(jnp.max(si, axis=0, keepdims=True), 0.0)
        ez = jnp.exp(si - zmax)
        den = jnp.sum(ez, axis=0, keepdims=True) + 1.0
        attn = (ez / den).astype(jnp.bfloat16)
        ai = activ_b[i * M:(i + 1) * M]
        ta = jax.lax.dot_general(attn, ai, (((0,), (0,)), ((), ())),
                                 preferred_element_type=jnp.float32)
        full = jnp.dot(ta.astype(jnp.bfloat16), Wv_ref[...],
                       preferred_element_type=jnp.float32)
        sa = ((den - 1.0) / den).reshape(H, 1)
        full = full + sa * bv_ref[...]
        out_ref[i] = jnp.sum(full * pick, axis=0, keepdims=True)
        frac_ref[i] = jnp.sum(m[i * M:(i + 1) * M], axis=0,
                              keepdims=True) * (1.0 / M)


@jax.jit
def kernel(x, Ws, bs, Wq, bq, Wk, bk, Wv, bv):
    xf = x[:, : M * D].reshape(B, M, D).astype(jnp.bfloat16)
    mask = x[:, M * D:].reshape(B, M, 1)
    out_main, frac = pl.pallas_call(
        _body,
        grid=(B // BPG,),
        in_specs=[
            pl.BlockSpec((BPG, M, D), lambda b: (b, 0, 0)),
            pl.BlockSpec((BPG, M, 1), lambda b: (b, 0, 0)),
            pl.BlockSpec((D, D), lambda b: (0, 0)),
            pl.BlockSpec((1, D), lambda b: (0, 0)),
            pl.BlockSpec((D, HA), lambda b: (0, 0)),
            pl.BlockSpec((1, HA), lambda b: (0, 0)),
            pl.BlockSpec((D, HA), lambda b: (0, 0)),
            pl.BlockSpec((1, HA), lambda b: (0, 0)),
            pl.BlockSpec((D, HO), lambda b: (0, 0)),
            pl.BlockSpec((1, HO), lambda b: (0, 0)),
        ],
        out_specs=[
            pl.BlockSpec((BPG, 1, HO), lambda b: (b, 0, 0)),
            pl.BlockSpec((BPG, 1, 1), lambda b: (b, 0, 0)),
        ],
        out_shape=[
            jax.ShapeDtypeStruct((B, 1, HO), jnp.float32),
            jax.ShapeDtypeStruct((B, 1, 1), jnp.float32),
        ],
    )(xf, mask, Ws.astype(jnp.bfloat16), bs.reshape(1, D),
      Wq.astype(jnp.bfloat16), bq.reshape(1, HA),
      Wk.astype(jnp.bfloat16), bk.reshape(1, HA),
      Wv.astype(jnp.bfloat16), bv.reshape(1, HO))
    return jnp.concatenate([out_main.reshape(B, HO), frac.reshape(B, 1)],
                           axis=1)
